# Initial kernel scaffold; baseline (speedup 1.0000x reference)
import jax
import jax.numpy as jnp
from jax.experimental import pallas as pl


def kernel(x, edge_index, edge_attr, params):
    raise NotImplementedError("write your pallas kernel here")


# speedup vs baseline: 3.2561x; 3.2561x over previous
import functools

import jax
import jax.numpy as jnp
from jax import lax
from jax.experimental import pallas as pl
from jax.experimental.pallas import tpu as pltpu
from jax.experimental.pallas import tpu_sc as plsc

N = 10000
E = 320000
D = 128

NC = 2
NS = 16
NW = NC * NS
EPW = E // NW
CHUNK = 80
NCHUNK = EPW // CHUNK
NPAD = 10240
RPT = NPAD // NS

def _gather_sc_body(src_hbm, dst_hbm, xs_hbm, xd_hbm, outs_hbm, outd_hbm,
                    idx_s, idx_d, rows_s, rows_d, sem_s, sem_d):
    wid = lax.axis_index("s") * NC + lax.axis_index("c")
    base = wid * EPW
    pltpu.sync_copy(src_hbm.at[wid], idx_s)
    pltpu.sync_copy(dst_hbm.at[wid], idx_d)

    def body(k, carry):
        off = base + k * CHUNK
        cp_s = pltpu.async_copy(xs_hbm.at[idx_s.at[k]], rows_s, sem_s)
        cp_d = pltpu.async_copy(xd_hbm.at[idx_d.at[k]], rows_d, sem_d)
        cp_s.wait()
        cp_d.wait()
        pltpu.sync_copy(rows_s, outs_hbm.at[pl.ds(off, CHUNK)])
        pltpu.sync_copy(rows_d, outd_hbm.at[pl.ds(off, CHUNK)])
        return carry

    lax.fori_loop(0, NCHUNK, body, 0)


def _scatter_sc_body(dst_hbm, ea_hbm, zeros_hbm, out_hbm, idx_v, rows_v, acc,
                     sem):
    c = lax.axis_index("c")
    s = lax.axis_index("s")
    row0 = s * RPT
    pltpu.sync_copy(zeros_hbm, acc.at[pl.ds(row0, RPT)])
    wid = c * NS + s
    base = wid * EPW
    pltpu.sync_copy(dst_hbm.at[wid], idx_v)
    plsc.subcore_barrier()

    def body(k, carry):
        off = base + k * CHUNK
        pltpu.sync_copy(ea_hbm.at[pl.ds(off, CHUNK)], rows_v)
        pltpu.sync_copy(rows_v, acc.at[idx_v.at[k]], add=True)
        return carry

    lax.fori_loop(0, NCHUNK, body, 0)
    plsc.subcore_barrier()
    pltpu.sync_copy(acc.at[pl.ds(row0, RPT)], out_hbm.at[c, pl.ds(row0, RPT)])


@functools.cache
def _sc_kernels():
    mesh = plsc.VectorSubcoreMesh(core_axis_name="c", subcore_axis_name="s",
                                  num_cores=NC, num_subcores=NS)
    gather = pl.kernel(
        _gather_sc_body,
        out_type=(jax.ShapeDtypeStruct((E, D), jnp.float32),
                  jax.ShapeDtypeStruct((E, D), jnp.float32)),
        mesh=mesh,
        scratch_types=[
            pltpu.VMEM((NCHUNK, CHUNK), jnp.int32),
            pltpu.VMEM((NCHUNK, CHUNK), jnp.int32),
            pltpu.VMEM((CHUNK, D), jnp.float32),
            pltpu.VMEM((CHUNK, D), jnp.float32),
            pltpu.SemaphoreType.DMA,
            pltpu.SemaphoreType.DMA,
        ],
    )
    scatter = pl.kernel(
        _scatter_sc_body,
        out_type=jax.ShapeDtypeStruct((NC, NPAD, D), jnp.float32),
        mesh=mesh,
        scratch_types=[
            pltpu.VMEM((NCHUNK, CHUNK), jnp.int32),
            pltpu.VMEM((CHUNK, D), jnp.float32),
            pltpu.VMEM_SHARED((NPAD, D), jnp.float32),
            pltpu.SemaphoreType.DMA,
        ],
    )
    return gather, scatter


def _gather_sc(src, dst, xs, xd):
    return _sc_kernels()[0](src, dst, xs, xd)


def _scatter_sc(dst, ea, zeros):
    return _sc_kernels()[1](dst, ea, zeros)


def _full(shape):
    return pl.BlockSpec(shape, lambda i: (0,) * len(shape))


def _xsxd_body(x_ref, w1s_ref, w1d_ref, b1_ref, xs_ref, xd_ref):
    x = x_ref[...]
    xs_ref[...] = jnp.dot(x, w1s_ref[...], preferred_element_type=jnp.float32)
    xd_ref[...] = (jnp.dot(x, w1d_ref[...], preferred_element_type=jnp.float32)
                   + b1_ref[...])


def _ln(h, g, b):
    mu = jnp.mean(h, axis=-1, keepdims=True)
    hc = h - mu
    var = jnp.mean(hc * hc, axis=-1, keepdims=True)
    return hc * lax.rsqrt(var + 1e-5) * g + b


def _edge_body(gs_ref, gd_ref, ea_ref, w1e_ref, w2_ref, b2_ref, w3_ref,
               b3_ref, g_ref, b_ref, out_ref):
    ea = ea_ref[...]
    h = (gs_ref[...] + gd_ref[...]
         + jnp.dot(ea, w1e_ref[...], preferred_element_type=jnp.float32))
    h = jnp.maximum(h, 0.0)
    h = jnp.dot(h, w2_ref[...], preferred_element_type=jnp.float32) + b2_ref[...]
    h = jnp.maximum(h, 0.0)
    h = jnp.dot(h, w3_ref[...], preferred_element_type=jnp.float32) + b3_ref[...]
    out_ref[...] = _ln(h, g_ref[...], b_ref[...]) + ea


def _node_body(x_ref, agg_ref, w1x_ref, w1a_ref, b1_ref, w2_ref, b2_ref,
               w3_ref, b3_ref, g_ref, b_ref, out_ref):
    x = x_ref[...]
    agg = agg_ref[0] + agg_ref[1]
    h = (jnp.dot(x, w1x_ref[...], preferred_element_type=jnp.float32)
         + jnp.dot(agg, w1a_ref[...], preferred_element_type=jnp.float32)
         + b1_ref[...])
    h = jnp.maximum(h, 0.0)
    h = jnp.dot(h, w2_ref[...], preferred_element_type=jnp.float32) + b2_ref[...]
    h = jnp.maximum(h, 0.0)
    h = jnp.dot(h, w3_ref[...], preferred_element_type=jnp.float32) + b3_ref[...]
    out_ref[...] = _ln(h, g_ref[...], b_ref[...]) + x


TN = 1000
TE = 2000


def _xsxd_call(x, w1s, w1d, b1):
    return pl.pallas_call(
        _xsxd_body,
        grid=(N // TN,),
        in_specs=[pl.BlockSpec((TN, D), lambda i: (i, 0)),
                  _full((D, D)), _full((D, D)), _full((1, D))],
        out_specs=[pl.BlockSpec((TN, D), lambda i: (i, 0))] * 2,
        out_shape=[jax.ShapeDtypeStruct((N, D), jnp.float32)] * 2,
    )(x, w1s, w1d, b1)


def _edge_call(gs, gd, ea, w1e, w2, b2, w3, b3, g, b):
    row = pl.BlockSpec((TE, D), lambda i: (i, 0))
    return pl.pallas_call(
        _edge_body,
        grid=(E // TE,),
        in_specs=[row, row, row, _full((D, D)), _full((D, D)), _full((1, D)),
                  _full((D, D)), _full((1, D)), _full((1, D)), _full((1, D))],
        out_specs=row,
        out_shape=jax.ShapeDtypeStruct((E, D), jnp.float32),
    )(gs, gd, ea, w1e, w2, b2, w3, b3, g, b)


def _node_call(x, agg2, w1x, w1a, b1, w2, b2, w3, b3, g, b):
    row = pl.BlockSpec((TN, D), lambda i: (i, 0))
    return pl.pallas_call(
        _node_body,
        grid=(N // TN,),
        in_specs=[row, pl.BlockSpec((NC, TN, D), lambda i: (0, i, 0)),
                  _full((D, D)), _full((D, D)), _full((1, D)),
                  _full((D, D)), _full((1, D)),
                  _full((D, D)), _full((1, D)), _full((1, D)), _full((1, D))],
        out_specs=row,
        out_shape=jax.ShapeDtypeStruct((N, D), jnp.float32),
    )(x, agg2, w1x, w1a, b1, w2, b2, w3, b3, g, b)


def kernel(x, edge_index, edge_attr, params):
    src = edge_index[0].reshape(NW, NCHUNK, CHUNK)
    dst = edge_index[1].reshape(NW, NCHUNK, CHUNK)
    zeros = jnp.zeros((RPT, D), jnp.float32)
    r = lambda v: v.reshape(1, D)
    for blk in params:
        (W1, b1), (W2, b2), (W3, b3) = blk["edge"]["linears"]
        xs, xd = _xsxd_call(x, W1[:D], W1[D:2 * D], r(b1))
        gs, gd = _gather_sc(src, dst, xs, xd)
        edge_attr = _edge_call(gs, gd, edge_attr, W1[2 * D:], W2, r(b2),
                               W3, r(b3), r(blk["edge"]["ln_g"]),
                               r(blk["edge"]["ln_b"]))
        agg2 = _scatter_sc(dst, edge_attr, zeros)
        (V1, c1), (V2, c2), (V3, c3) = blk["node"]["linears"]
        x = _node_call(x, agg2, V1[:D], V1[D:], r(c1), V2, r(c2), V3, r(c3),
                       r(blk["node"]["ln_g"]), r(blk["node"]["ln_b"]))
    return x


# speedup vs baseline: 3.4027x; 1.0450x over previous
import functools

import jax
import jax.numpy as jnp
from jax import lax
from jax.experimental import pallas as pl
from jax.experimental.pallas import tpu as pltpu
from jax.experimental.pallas import tpu_sc as plsc

N = 10000
E = 320000
D = 128

NC = 2
NS = 16
NW = NC * NS
EPW = E // NW
CHUNK = 80
NCHUNK = EPW // CHUNK
NPAD = 10240
RPT = NPAD // NS


def _gather_sc_body(src_hbm, dst_hbm, xs_hbm, xd_hbm, outs_hbm, outd_hbm,
                    idx_s, idx_d, rows_s, rows_d, sem_s, sem_d):
    wid = lax.axis_index("s") * NC + lax.axis_index("c")
    base = wid * EPW
    pltpu.sync_copy(src_hbm.at[wid], idx_s)
    pltpu.sync_copy(dst_hbm.at[wid], idx_d)

    def body(k, carry):
        off = base + k * CHUNK
        cp_s = pltpu.async_copy(xs_hbm.at[idx_s.at[k]], rows_s, sem_s)
        cp_d = pltpu.async_copy(xd_hbm.at[idx_d.at[k]], rows_d, sem_d)
        cp_s.wait()
        cp_d.wait()
        pltpu.sync_copy(rows_s, outs_hbm.at[pl.ds(off, CHUNK)])
        pltpu.sync_copy(rows_d, outd_hbm.at[pl.ds(off, CHUNK)])
        return carry

    lax.fori_loop(0, NCHUNK, body, 0)


def _scatter_sc_body(dst_hbm, ea_hbm, zeros_hbm, out_hbm, idx_v, rows_a,
                     rows_b, acc, seml_a, seml_b):
    c = lax.axis_index("c")
    s = lax.axis_index("s")
    row0 = s * RPT
    pltpu.sync_copy(zeros_hbm, acc.at[pl.ds(row0, RPT)])
    wid = c * NS + s
    base = wid * EPW
    pltpu.sync_copy(dst_hbm.at[wid], idx_v)
    plsc.subcore_barrier()

    def load(k, buf, seml):
        pltpu.async_copy(ea_hbm.at[pl.ds(base + k * CHUNK, CHUNK)], buf, seml)

    def drain(buf, seml):
        pltpu.make_async_copy(ea_hbm.at[pl.ds(base, CHUNK)], buf, seml).wait()

    load(0, rows_a, seml_a)

    def body(t, carry):
        k = 2 * t
        drain(rows_a, seml_a)
        load(k + 1, rows_b, seml_b)
        pltpu.sync_copy(rows_a, acc.at[idx_v.at[k]], add=True)
        drain(rows_b, seml_b)
        load(k + 2, rows_a, seml_a)
        pltpu.sync_copy(rows_b, acc.at[idx_v.at[k + 1]], add=True)
        return carry

    lax.fori_loop(0, NCHUNK // 2, body, 0)
    drain(rows_a, seml_a)
    pltpu.sync_copy(rows_a, acc.at[idx_v.at[NCHUNK - 1]], add=True)
    plsc.subcore_barrier()
    pltpu.sync_copy(acc.at[pl.ds(row0, RPT)], out_hbm.at[c, pl.ds(row0, RPT)])


@functools.cache
def _sc_kernels():
    mesh = plsc.VectorSubcoreMesh(core_axis_name="c", subcore_axis_name="s",
                                  num_cores=NC, num_subcores=NS)
    gather = pl.kernel(
        _gather_sc_body,
        out_type=(jax.ShapeDtypeStruct((E, D), jnp.float32),
                  jax.ShapeDtypeStruct((E, D), jnp.float32)),
        mesh=mesh,
        scratch_types=[
            pltpu.VMEM((NCHUNK, CHUNK), jnp.int32),
            pltpu.VMEM((NCHUNK, CHUNK), jnp.int32),
            pltpu.VMEM((CHUNK, D), jnp.float32),
            pltpu.VMEM((CHUNK, D), jnp.float32),
            pltpu.SemaphoreType.DMA,
            pltpu.SemaphoreType.DMA,
        ],
    )
    scatter = pl.kernel(
        _scatter_sc_body,
        out_type=jax.ShapeDtypeStruct((NC, NPAD, D), jnp.float32),
        mesh=mesh,
        scratch_types=[
            pltpu.VMEM((NCHUNK, CHUNK), jnp.int32),
            pltpu.VMEM((CHUNK, D), jnp.float32),
            pltpu.VMEM((CHUNK, D), jnp.float32),
            pltpu.VMEM_SHARED((NPAD, D), jnp.float32),
            pltpu.SemaphoreType.DMA,
            pltpu.SemaphoreType.DMA,
        ],
    )
    return gather, scatter


def _gather_sc(src, dst, xs, xd):
    return _sc_kernels()[0](src, dst, xs, xd)


def _scatter_sc(dst, ea, zeros):
    return _sc_kernels()[1](dst, ea, zeros)


def _full(shape):
    return pl.BlockSpec(shape, lambda i: (0,) * len(shape))


def _xsxd_body(x_ref, w1s_ref, w1d_ref, b1_ref, xs_ref, xd_ref):
    x = x_ref[...]
    xs_ref[...] = jnp.dot(x, w1s_ref[...], preferred_element_type=jnp.float32)
    xd_ref[...] = (jnp.dot(x, w1d_ref[...], preferred_element_type=jnp.float32)
                   + b1_ref[...])


def _ln(h, g, b):
    mu = jnp.mean(h, axis=-1, keepdims=True)
    hc = h - mu
    var = jnp.mean(hc * hc, axis=-1, keepdims=True)
    return hc * lax.rsqrt(var + 1e-5) * g + b


def _edge_body(gs_ref, gd_ref, ea_ref, w1e_ref, w2_ref, b2_ref, w3_ref,
               b3_ref, g_ref, b_ref, out_ref):
    ea = ea_ref[...]
    bf = jnp.bfloat16
    h = (gs_ref[...] + gd_ref[...]
         + jnp.dot(ea.astype(bf), w1e_ref[...].astype(bf),
                   preferred_element_type=jnp.float32))
    h = jnp.maximum(h, 0.0)
    h = jnp.dot(h.astype(bf), w2_ref[...].astype(bf),
                preferred_element_type=jnp.float32) + b2_ref[...]
    h = jnp.maximum(h, 0.0)
    h = jnp.dot(h.astype(bf), w3_ref[...].astype(bf),
                preferred_element_type=jnp.float32) + b3_ref[...]
    out_ref[...] = _ln(h, g_ref[...], b_ref[...]) + ea


def _node_body(x_ref, agg_ref, w1x_ref, w1a_ref, b1_ref, w2_ref, b2_ref,
               w3_ref, b3_ref, g_ref, b_ref, out_ref):
    x = x_ref[...]
    agg = agg_ref[0] + agg_ref[1]
    h = (jnp.dot(x, w1x_ref[...], preferred_element_type=jnp.float32)
         + jnp.dot(agg, w1a_ref[...], preferred_element_type=jnp.float32)
         + b1_ref[...])
    h = jnp.maximum(h, 0.0)
    h = jnp.dot(h, w2_ref[...], preferred_element_type=jnp.float32) + b2_ref[...]
    h = jnp.maximum(h, 0.0)
    h = jnp.dot(h, w3_ref[...], preferred_element_type=jnp.float32) + b3_ref[...]
    out_ref[...] = _ln(h, g_ref[...], b_ref[...]) + x


TN = 1000
TE = 2000


def _xsxd_call(x, w1s, w1d, b1):
    return pl.pallas_call(
        _xsxd_body,
        grid=(N // TN,),
        in_specs=[pl.BlockSpec((TN, D), lambda i: (i, 0)),
                  _full((D, D)), _full((D, D)), _full((1, D))],
        out_specs=[pl.BlockSpec((TN, D), lambda i: (i, 0))] * 2,
        out_shape=[jax.ShapeDtypeStruct((N, D), jnp.float32)] * 2,
    )(x, w1s, w1d, b1)


def _edge_call(gs, gd, ea, w1e, w2, b2, w3, b3, g, b):
    row = pl.BlockSpec((TE, D), lambda i: (i, 0))
    return pl.pallas_call(
        _edge_body,
        grid=(E // TE,),
        in_specs=[row, row, row, _full((D, D)), _full((D, D)), _full((1, D)),
                  _full((D, D)), _full((1, D)), _full((1, D)), _full((1, D))],
        out_specs=row,
        out_shape=jax.ShapeDtypeStruct((E, D), jnp.float32),
    )(gs, gd, ea, w1e, w2, b2, w3, b3, g, b)


def _node_call(x, agg2, w1x, w1a, b1, w2, b2, w3, b3, g, b):
    row = pl.BlockSpec((TN, D), lambda i: (i, 0))
    return pl.pallas_call(
        _node_body,
        grid=(N // TN,),
        in_specs=[row, pl.BlockSpec((NC, TN, D), lambda i: (0, i, 0)),
                  _full((D, D)), _full((D, D)), _full((1, D)),
                  _full((D, D)), _full((1, D)),
                  _full((D, D)), _full((1, D)), _full((1, D)), _full((1, D))],
        out_specs=row,
        out_shape=jax.ShapeDtypeStruct((N, D), jnp.float32),
    )(x, agg2, w1x, w1a, b1, w2, b2, w3, b3, g, b)


def kernel(x, edge_index, edge_attr, params):
    src = edge_index[0].reshape(NW, NCHUNK, CHUNK)
    dst = edge_index[1].reshape(NW, NCHUNK, CHUNK)
    zeros = jnp.zeros((RPT, D), jnp.float32)
    r = lambda v: v.reshape(1, D)
    for blk in params:
        (W1, b1), (W2, b2), (W3, b3) = blk["edge"]["linears"]
        xs, xd = _xsxd_call(x, W1[:D], W1[D:2 * D], r(b1))
        gs, gd = _gather_sc(src, dst, xs, xd)
        edge_attr = _edge_call(gs, gd, edge_attr, W1[2 * D:], W2, r(b2),
                               W3, r(b3), r(blk["edge"]["ln_g"]),
                               r(blk["edge"]["ln_b"]))
        agg2 = _scatter_sc(dst, edge_attr, zeros)
        (V1, c1), (V2, c2), (V3, c3) = blk["node"]["linears"]
        x = _node_call(x, agg2, V1[:D], V1[D:], r(c1), V2, r(c2), V3, r(c3),
                       r(blk["node"]["ln_g"]), r(blk["node"]["ln_b"]))
    return x


# speedup vs baseline: 4.0457x; 1.1890x over previous
import functools

import jax
import jax.numpy as jnp
from jax import lax
from jax.experimental import pallas as pl
from jax.experimental.pallas import tpu as pltpu
from jax.experimental.pallas import tpu_sc as plsc

N = 10000
E = 320000
D = 128

NC = 2
NS = 16
NW = NC * NS
CHUNK = 80
NPAD = 10240
RPT = NPAD // NS

EH = (192000, 128000)
EOFF = (0, 192000)


def _make_gather_body(epw, nchunk):
    def body_fn(src_hbm, dst_hbm, xs_hbm, xd_hbm, outs_hbm, outd_hbm,
                idx_s, idx_d, rows_s, rows_d, sem_s, sem_d):
        wid = lax.axis_index("s") * NC + lax.axis_index("c")
        base = wid * epw
        pltpu.sync_copy(src_hbm.at[wid], idx_s)
        pltpu.sync_copy(dst_hbm.at[wid], idx_d)

        def body(k, carry):
            off = base + k * CHUNK
            cp_s = pltpu.async_copy(xs_hbm.at[idx_s.at[k]], rows_s, sem_s)
            cp_d = pltpu.async_copy(xd_hbm.at[idx_d.at[k]], rows_d, sem_d)
            cp_s.wait()
            cp_d.wait()
            pltpu.sync_copy(rows_s, outs_hbm.at[pl.ds(off, CHUNK)])
            pltpu.sync_copy(rows_d, outd_hbm.at[pl.ds(off, CHUNK)])
            return carry

        lax.fori_loop(0, nchunk, body, 0)

    return body_fn


def _make_scatter_body(epw, nchunk):
    def body_fn(dst_hbm, ea_hbm, zeros_hbm, out_hbm, idx_v, rows_a,
                rows_b, acc, seml_a, seml_b):
        c = lax.axis_index("c")
        s = lax.axis_index("s")
        row0 = s * RPT
        pltpu.sync_copy(zeros_hbm, acc.at[pl.ds(row0, RPT)])
        wid = c * NS + s
        base = wid * epw
        pltpu.sync_copy(dst_hbm.at[wid], idx_v)
        plsc.subcore_barrier()

        def load(k, buf, seml):
            pltpu.async_copy(ea_hbm.at[pl.ds(base + k * CHUNK, CHUNK)], buf,
                             seml)

        def drain(buf, seml):
            pltpu.make_async_copy(ea_hbm.at[pl.ds(base, CHUNK)], buf,
                                  seml).wait()

        def scat(k, buf):
            pltpu.sync_copy(buf, acc.at[idx_v.at[k]], add=True)

        npairs = (nchunk - 1) // 2
        load(0, rows_a, seml_a)

        def body(t, carry):
            k = 2 * t
            drain(rows_a, seml_a)
            load(k + 1, rows_b, seml_b)
            scat(k, rows_a)
            drain(rows_b, seml_b)
            load(k + 2, rows_a, seml_a)
            scat(k + 1, rows_b)
            return carry

        lax.fori_loop(0, npairs, body, 0)
        if nchunk % 2:
            drain(rows_a, seml_a)
            scat(nchunk - 1, rows_a)
        else:
            drain(rows_a, seml_a)
            load(nchunk - 1, rows_b, seml_b)
            scat(nchunk - 2, rows_a)
            drain(rows_b, seml_b)
            scat(nchunk - 1, rows_b)
        plsc.subcore_barrier()
        pltpu.sync_copy(acc.at[pl.ds(row0, RPT)],
                        out_hbm.at[c, pl.ds(row0, RPT)])

    return body_fn


@functools.cache
def _sc_kernels(eh):
    epw = eh // NW
    nchunk = epw // CHUNK
    mesh = plsc.VectorSubcoreMesh(core_axis_name="c", subcore_axis_name="s",
                                  num_cores=NC, num_subcores=NS)
    gather = pl.kernel(
        _make_gather_body(epw, nchunk),
        out_type=(jax.ShapeDtypeStruct((eh, D), jnp.float32),
                  jax.ShapeDtypeStruct((eh, D), jnp.float32)),
        mesh=mesh,
        scratch_types=[
            pltpu.VMEM((nchunk, CHUNK), jnp.int32),
            pltpu.VMEM((nchunk, CHUNK), jnp.int32),
            pltpu.VMEM((CHUNK, D), jnp.float32),
            pltpu.VMEM((CHUNK, D), jnp.float32),
            pltpu.SemaphoreType.DMA,
            pltpu.SemaphoreType.DMA,
        ],
    )
    scatter = pl.kernel(
        _make_scatter_body(epw, nchunk),
        out_type=jax.ShapeDtypeStruct((NC, NPAD, D), jnp.float32),
        mesh=mesh,
        scratch_types=[
            pltpu.VMEM((nchunk, CHUNK), jnp.int32),
            pltpu.VMEM((CHUNK, D), jnp.float32),
            pltpu.VMEM((CHUNK, D), jnp.float32),
            pltpu.VMEM_SHARED((NPAD, D), jnp.float32),
            pltpu.SemaphoreType.DMA,
            pltpu.SemaphoreType.DMA,
        ],
    )
    return gather, scatter


def _gather_sc(h, src, dst, xs, xd):
    return _sc_kernels(EH[h])[0](src, dst, xs, xd)


def _scatter_sc(h, dst, ea, zeros):
    return _sc_kernels(EH[h])[1](dst, ea, zeros)


def _full(shape):
    return pl.BlockSpec(shape, lambda i: (0,) * len(shape))


def _xsxd_body(x_ref, w1s_ref, w1d_ref, b1_ref, xs_ref, xd_ref):
    x = x_ref[...]
    xs_ref[...] = jnp.dot(x, w1s_ref[...], preferred_element_type=jnp.float32)
    xd_ref[...] = (jnp.dot(x, w1d_ref[...], preferred_element_type=jnp.float32)
                   + b1_ref[...])


def _ln(h, g, b):
    mu = jnp.mean(h, axis=-1, keepdims=True)
    hc = h - mu
    var = jnp.mean(hc * hc, axis=-1, keepdims=True)
    return hc * lax.rsqrt(var + 1e-5) * g + b


def _edge_body(gs_ref, gd_ref, ea_ref, w1e_ref, w2_ref, b2_ref, w3_ref,
               b3_ref, g_ref, b_ref, out_ref):
    ea = ea_ref[...]
    bf = jnp.bfloat16
    h = (gs_ref[...] + gd_ref[...]
         + jnp.dot(ea.astype(bf), w1e_ref[...].astype(bf),
                   preferred_element_type=jnp.float32))
    h = jnp.maximum(h, 0.0)
    h = jnp.dot(h.astype(bf), w2_ref[...].astype(bf),
                preferred_element_type=jnp.float32) + b2_ref[...]
    h = jnp.maximum(h, 0.0)
    h = jnp.dot(h.astype(bf), w3_ref[...].astype(bf),
                preferred_element_type=jnp.float32) + b3_ref[...]
    out_ref[...] = _ln(h, g_ref[...], b_ref[...]) + ea


def _node_body(x_ref, p0_ref, p1_ref, w1x_ref, w1a_ref, b1_ref, w2_ref,
               b2_ref, w3_ref, b3_ref, g_ref, b_ref, out_ref):
    x = x_ref[...]
    agg = (p0_ref[0] + p0_ref[1]) + (p1_ref[0] + p1_ref[1])
    h = (jnp.dot(x, w1x_ref[...], preferred_element_type=jnp.float32)
         + jnp.dot(agg, w1a_ref[...], preferred_element_type=jnp.float32)
         + b1_ref[...])
    h = jnp.maximum(h, 0.0)
    h = jnp.dot(h, w2_ref[...], preferred_element_type=jnp.float32) + b2_ref[...]
    h = jnp.maximum(h, 0.0)
    h = jnp.dot(h, w3_ref[...], preferred_element_type=jnp.float32) + b3_ref[...]
    out_ref[...] = _ln(h, g_ref[...], b_ref[...]) + x


TN = 1000
TE = 2000


def _xsxd_call(x, w1s, w1d, b1):
    return pl.pallas_call(
        _xsxd_body,
        grid=(N // TN,),
        in_specs=[pl.BlockSpec((TN, D), lambda i: (i, 0)),
                  _full((D, D)), _full((D, D)), _full((1, D))],
        out_specs=[pl.BlockSpec((TN, D), lambda i: (i, 0))] * 2,
        out_shape=[jax.ShapeDtypeStruct((N, D), jnp.float32)] * 2,
    )(x, w1s, w1d, b1)


def _edge_call(eh, ea_off, gs, gd, ea, w1e, w2, b2, w3, b3, g, b):
    row = pl.BlockSpec((TE, D), lambda i: (i, 0))
    ob = ea_off // TE
    row_ea = pl.BlockSpec((TE, D), lambda i: (ob + i, 0))
    return pl.pallas_call(
        _edge_body,
        grid=(eh // TE,),
        in_specs=[row, row, row_ea, _full((D, D)), _full((D, D)),
                  _full((1, D)), _full((D, D)), _full((1, D)), _full((1, D)),
                  _full((1, D))],
        out_specs=row,
        out_shape=jax.ShapeDtypeStruct((eh, D), jnp.float32),
    )(gs, gd, ea, w1e, w2, b2, w3, b3, g, b)


def _node_call(x, p0, p1, w1x, w1a, b1, w2, b2, w3, b3, g, b):
    row = pl.BlockSpec((TN, D), lambda i: (i, 0))
    agg_spec = pl.BlockSpec((NC, TN, D), lambda i: (0, i, 0))
    return pl.pallas_call(
        _node_body,
        grid=(N // TN,),
        in_specs=[row, agg_spec, agg_spec,
                  _full((D, D)), _full((D, D)), _full((1, D)),
                  _full((D, D)), _full((1, D)),
                  _full((D, D)), _full((1, D)), _full((1, D)), _full((1, D))],
        out_specs=row,
        out_shape=jax.ShapeDtypeStruct((N, D), jnp.float32),
    )(x, p0, p1, w1x, w1a, b1, w2, b2, w3, b3, g, b)


def kernel(x, edge_index, edge_attr, params):
    src_h, dst_h = [], []
    for h in range(2):
        epw = EH[h] // NW
        nch = epw // CHUNK
        sl = edge_index[:, EOFF[h]:EOFF[h] + EH[h]]
        src_h.append(sl[0].reshape(NW, nch, CHUNK))
        dst_h.append(sl[1].reshape(NW, nch, CHUNK))
    zeros = jnp.zeros((RPT, D), jnp.float32)
    r = lambda v: v.reshape(1, D)
    ea = (edge_attr, edge_attr)
    ea_off = (EOFF[0], EOFF[1])
    for blk in params:
        (W1, b1), (W2, b2), (W3, b3) = blk["edge"]["linears"]
        xs, xd = _xsxd_call(x, W1[:D], W1[D:2 * D], r(b1))
        g0 = _gather_sc(0, src_h[0], dst_h[0], xs, xd)
        g1 = _gather_sc(1, src_h[1], dst_h[1], xs, xd)
        eargs = (W1[2 * D:], W2, r(b2), W3, r(b3), r(blk["edge"]["ln_g"]),
                 r(blk["edge"]["ln_b"]))
        ea0 = _edge_call(EH[0], ea_off[0], g0[0], g0[1], ea[0], *eargs)
        p0 = _scatter_sc(0, dst_h[0], ea0, zeros)
        ea1 = _edge_call(EH[1], ea_off[1], g1[0], g1[1], ea[1], *eargs)
        p1 = _scatter_sc(1, dst_h[1], ea1, zeros)
        ea, ea_off = (ea0, ea1), (0, 0)
        (V1, c1), (V2, c2), (V3, c3) = blk["node"]["linears"]
        x = _node_call(x, p0, p1, V1[:D], V1[D:], r(c1), V2, r(c2), V3, r(c3),
                       r(blk["node"]["ln_g"]), r(blk["node"]["ln_b"]))
    return x


# speedup vs baseline: 4.0988x; 1.0131x over previous
import functools

import jax
import jax.numpy as jnp
from jax import lax
from jax.experimental import pallas as pl
from jax.experimental.pallas import tpu as pltpu
from jax.experimental.pallas import tpu_sc as plsc

N = 10000
E = 320000
D = 128

NC = 2
NS = 16
NW = NC * NS
CHUNK = 80
NPAD = 10240
RPT = NPAD // NS

EH = (192000, 128000)
EOFF = (0, 192000)


def _make_gather_body(epw, nchunk):
    def body_fn(src_hbm, dst_hbm, xs_hbm, xd_hbm, outs_hbm, outd_hbm,
                idx_s, idx_d, rows_s, rows_d, sem_s, sem_d):
        wid = lax.axis_index("s") * NC + lax.axis_index("c")
        base = wid * epw
        pltpu.sync_copy(src_hbm.at[wid], idx_s)
        pltpu.sync_copy(dst_hbm.at[wid], idx_d)

        def body(k, carry):
            off = base + k * CHUNK
            cp_s = pltpu.async_copy(xs_hbm.at[idx_s.at[k]], rows_s, sem_s)
            cp_d = pltpu.async_copy(xd_hbm.at[idx_d.at[k]], rows_d, sem_d)
            cp_s.wait()
            cp_d.wait()
            pltpu.sync_copy(rows_s, outs_hbm.at[pl.ds(off, CHUNK)])
            pltpu.sync_copy(rows_d, outd_hbm.at[pl.ds(off, CHUNK)])
            return carry

        lax.fori_loop(0, nchunk, body, 0)

    return body_fn


def _make_scatter_body(epw, nchunk):
    def body_fn(dst_hbm, ea_hbm, zeros_hbm, out_hbm, idx_v, rows_a,
                rows_b, acc, seml_a, seml_b):
        c = lax.axis_index("c")
        s = lax.axis_index("s")
        row0 = s * RPT
        pltpu.sync_copy(zeros_hbm, acc.at[pl.ds(row0, RPT)])
        wid = c * NS + s
        base = wid * epw
        pltpu.sync_copy(dst_hbm.at[wid], idx_v)
        plsc.subcore_barrier()

        def load(k, buf, seml):
            pltpu.async_copy(ea_hbm.at[pl.ds(base + k * CHUNK, CHUNK)], buf,
                             seml)

        def drain(buf, seml):
            pltpu.make_async_copy(ea_hbm.at[pl.ds(base, CHUNK)], buf,
                                  seml).wait()

        def scat(k, buf):
            pltpu.sync_copy(buf, acc.at[idx_v.at[k]], add=True)

        npairs = (nchunk - 1) // 2
        load(0, rows_a, seml_a)

        def body(t, carry):
            k = 2 * t
            drain(rows_a, seml_a)
            load(k + 1, rows_b, seml_b)
            scat(k, rows_a)
            drain(rows_b, seml_b)
            load(k + 2, rows_a, seml_a)
            scat(k + 1, rows_b)
            return carry

        lax.fori_loop(0, npairs, body, 0)
        if nchunk % 2:
            drain(rows_a, seml_a)
            scat(nchunk - 1, rows_a)
        else:
            drain(rows_a, seml_a)
            load(nchunk - 1, rows_b, seml_b)
            scat(nchunk - 2, rows_a)
            drain(rows_b, seml_b)
            scat(nchunk - 1, rows_b)
        plsc.subcore_barrier()
        pltpu.sync_copy(acc.at[pl.ds(row0, RPT)],
                        out_hbm.at[c, pl.ds(row0, RPT)])

    return body_fn


@functools.cache
def _sc_kernels(eh):
    epw = eh // NW
    nchunk = epw // CHUNK
    mesh = plsc.VectorSubcoreMesh(core_axis_name="c", subcore_axis_name="s",
                                  num_cores=NC, num_subcores=NS)
    gather = pl.kernel(
        _make_gather_body(epw, nchunk),
        out_type=(jax.ShapeDtypeStruct((eh, D), jnp.float32),
                  jax.ShapeDtypeStruct((eh, D), jnp.float32)),
        mesh=mesh,
        scratch_types=[
            pltpu.VMEM((nchunk, CHUNK), jnp.int32),
            pltpu.VMEM((nchunk, CHUNK), jnp.int32),
            pltpu.VMEM((CHUNK, D), jnp.float32),
            pltpu.VMEM((CHUNK, D), jnp.float32),
            pltpu.SemaphoreType.DMA,
            pltpu.SemaphoreType.DMA,
        ],
    )
    scatter = pl.kernel(
        _make_scatter_body(epw, nchunk),
        out_type=jax.ShapeDtypeStruct((NC, NPAD, D), jnp.float32),
        mesh=mesh,
        scratch_types=[
            pltpu.VMEM((nchunk, CHUNK), jnp.int32),
            pltpu.VMEM((CHUNK, D), jnp.float32),
            pltpu.VMEM((CHUNK, D), jnp.float32),
            pltpu.VMEM_SHARED((NPAD, D), jnp.float32),
            pltpu.SemaphoreType.DMA,
            pltpu.SemaphoreType.DMA,
        ],
    )
    return gather, scatter


def _gather_sc(h, src, dst, xs, xd):
    return _sc_kernels(EH[h])[0](src, dst, xs, xd)


def _scatter_sc(h, dst, ea, zeros):
    return _sc_kernels(EH[h])[1](dst, ea, zeros)


def _full(shape):
    return pl.BlockSpec(shape, lambda i: (0,) * len(shape))


def _xsxd_body(x_ref, w1s_ref, w1d_ref, b1_ref, xs_ref, xd_ref):
    x = x_ref[...]
    xs_ref[...] = jnp.dot(x, w1s_ref[...], preferred_element_type=jnp.float32)
    xd_ref[...] = (jnp.dot(x, w1d_ref[...], preferred_element_type=jnp.float32)
                   + b1_ref[...])


def _ln(h, g, b):
    mu = jnp.mean(h, axis=-1, keepdims=True)
    hc = h - mu
    var = jnp.mean(hc * hc, axis=-1, keepdims=True)
    return hc * lax.rsqrt(var + 1e-5) * g + b


def _edge_body(gs_ref, gd_ref, ea_ref, w1e_ref, w2_ref, b2_ref, w3_ref,
               b3_ref, g_ref, b_ref, out_ref):
    ea = ea_ref[...]
    bf = jnp.bfloat16
    h = (gs_ref[...] + gd_ref[...]
         + jnp.dot(ea.astype(bf), w1e_ref[...].astype(bf),
                   preferred_element_type=jnp.float32))
    h = jnp.maximum(h, 0.0)
    h = jnp.dot(h.astype(bf), w2_ref[...].astype(bf),
                preferred_element_type=jnp.float32) + b2_ref[...]
    h = jnp.maximum(h, 0.0)
    h = jnp.dot(h.astype(bf), w3_ref[...].astype(bf),
                preferred_element_type=jnp.float32) + b3_ref[...]
    out_ref[...] = _ln(h, g_ref[...], b_ref[...]) + ea


def _node_body(x_ref, p0_ref, p1_ref, w1x_ref, w1a_ref, b1_ref, w2_ref,
               b2_ref, w3_ref, b3_ref, g_ref, b_ref, out_ref):
    x = x_ref[...]
    agg = (p0_ref[0] + p0_ref[1]) + (p1_ref[0] + p1_ref[1])
    h = (jnp.dot(x, w1x_ref[...], preferred_element_type=jnp.float32)
         + jnp.dot(agg, w1a_ref[...], preferred_element_type=jnp.float32)
         + b1_ref[...])
    h = jnp.maximum(h, 0.0)
    h = jnp.dot(h, w2_ref[...], preferred_element_type=jnp.float32) + b2_ref[...]
    h = jnp.maximum(h, 0.0)
    h = jnp.dot(h, w3_ref[...], preferred_element_type=jnp.float32) + b3_ref[...]
    out_ref[...] = _ln(h, g_ref[...], b_ref[...]) + x


TN = 1000
TE = 2560


def _xsxd_call(x, w1s, w1d, b1):
    return pl.pallas_call(
        _xsxd_body,
        grid=(N // TN,),
        in_specs=[pl.BlockSpec((TN, D), lambda i: (i, 0)),
                  _full((D, D)), _full((D, D)), _full((1, D))],
        out_specs=[pl.BlockSpec((TN, D), lambda i: (i, 0))] * 2,
        out_shape=[jax.ShapeDtypeStruct((N, D), jnp.float32)] * 2,
    )(x, w1s, w1d, b1)


def _edge_call(eh, ea_off, gs, gd, ea, w1e, w2, b2, w3, b3, g, b):
    row = pl.BlockSpec((TE, D), lambda i: (i, 0))
    ob = ea_off // TE
    row_ea = pl.BlockSpec((TE, D), lambda i: (ob + i, 0))
    return pl.pallas_call(
        _edge_body,
        grid=(eh // TE,),
        in_specs=[row, row, row_ea, _full((D, D)), _full((D, D)),
                  _full((1, D)), _full((D, D)), _full((1, D)), _full((1, D)),
                  _full((1, D))],
        out_specs=row,
        out_shape=jax.ShapeDtypeStruct((eh, D), jnp.float32),
    )(gs, gd, ea, w1e, w2, b2, w3, b3, g, b)


def _node_call(x, p0, p1, w1x, w1a, b1, w2, b2, w3, b3, g, b):
    row = pl.BlockSpec((TN, D), lambda i: (i, 0))
    agg_spec = pl.BlockSpec((NC, TN, D), lambda i: (0, i, 0))
    return pl.pallas_call(
        _node_body,
        grid=(N // TN,),
        in_specs=[row, agg_spec, agg_spec,
                  _full((D, D)), _full((D, D)), _full((1, D)),
                  _full((D, D)), _full((1, D)),
                  _full((D, D)), _full((1, D)), _full((1, D)), _full((1, D))],
        out_specs=row,
        out_shape=jax.ShapeDtypeStruct((N, D), jnp.float32),
    )(x, p0, p1, w1x, w1a, b1, w2, b2, w3, b3, g, b)


def kernel(x, edge_index, edge_attr, params):
    src_h, dst_h = [], []
    for h in range(2):
        epw = EH[h] // NW
        nch = epw // CHUNK
        sl = edge_index[:, EOFF[h]:EOFF[h] + EH[h]]
        src_h.append(sl[0].reshape(NW, nch, CHUNK))
        dst_h.append(sl[1].reshape(NW, nch, CHUNK))
    zeros = jnp.zeros((RPT, D), jnp.float32)
    r = lambda v: v.reshape(1, D)
    ea = (edge_attr, edge_attr)
    ea_off = (EOFF[0], EOFF[1])
    for blk in params:
        (W1, b1), (W2, b2), (W3, b3) = blk["edge"]["linears"]
        xs, xd = _xsxd_call(x, W1[:D], W1[D:2 * D], r(b1))
        g0 = _gather_sc(0, src_h[0], dst_h[0], xs, xd)
        g1 = _gather_sc(1, src_h[1], dst_h[1], xs, xd)
        eargs = (W1[2 * D:], W2, r(b2), W3, r(b3), r(blk["edge"]["ln_g"]),
                 r(blk["edge"]["ln_b"]))
        ea0 = _edge_call(EH[0], ea_off[0], g0[0], g0[1], ea[0], *eargs)
        p0 = _scatter_sc(0, dst_h[0], ea0, zeros)
        ea1 = _edge_call(EH[1], ea_off[1], g1[0], g1[1], ea[1], *eargs)
        p1 = _scatter_sc(1, dst_h[1], ea1, zeros)
        ea, ea_off = (ea0, ea1), (0, 0)
        (V1, c1), (V2, c2), (V3, c3) = blk["node"]["linears"]
        x = _node_call(x, p0, p1, V1[:D], V1[D:], r(c1), V2, r(c2), V3, r(c3),
                       r(blk["node"]["ln_g"]), r(blk["node"]["ln_b"]))
    return x


# speedup vs baseline: 4.2345x; 1.0331x over previous
import functools

import jax
import jax.numpy as jnp
from jax import lax
from jax.experimental import pallas as pl
from jax.experimental.pallas import tpu as pltpu
from jax.experimental.pallas import tpu_sc as plsc

N = 10000
E = 320000
D = 128

NC = 2
NS = 16
NW = NC * NS
CHUNK = 80
NPAD = 10240
RPT = NPAD // NS

EH = (192000, 128000)
EOFF = (0, 192000)


def _make_gather_body(epw, nchunk):
    def body_fn(src_hbm, dst_hbm, xs_hbm, xd_hbm, outs_hbm, outd_hbm,
                idx_s, idx_d, rs_a, rd_a, rs_b, rd_b,
                semg_a, semg_b, semw_a, semw_b):
        wid = lax.axis_index("s") * NC + lax.axis_index("c")
        base = wid * epw
        pltpu.sync_copy(src_hbm.at[wid], idx_s)
        pltpu.sync_copy(dst_hbm.at[wid], idx_d)

        A = (rs_a, rd_a, semg_a, semw_a)
        B = (rs_b, rd_b, semg_b, semw_b)

        def gath(k, buf):
            rs, rd, semg, _ = buf
            pltpu.async_copy(xs_hbm.at[idx_s.at[k]], rs, semg)
            pltpu.async_copy(xd_hbm.at[idx_d.at[k]], rd, semg)

        def drain_g(buf):
            rs, rd, semg, _ = buf
            pltpu.make_async_copy(xs_hbm.at[pl.ds(0, CHUNK)], rs, semg).wait()
            pltpu.make_async_copy(xs_hbm.at[pl.ds(0, CHUNK)], rd, semg).wait()

        def wrt(k, buf):
            rs, rd, _, semw = buf
            off = base + k * CHUNK
            pltpu.async_copy(rs, outs_hbm.at[pl.ds(off, CHUNK)], semw)
            pltpu.async_copy(rd, outd_hbm.at[pl.ds(off, CHUNK)], semw)

        def drain_w(buf):
            rs, rd, _, semw = buf
            pltpu.make_async_copy(outs_hbm.at[pl.ds(base, CHUNK)], rs,
                                  semw).wait()
            pltpu.make_async_copy(outs_hbm.at[pl.ds(base, CHUNK)], rd,
                                  semw).wait()

        wrt(1, B)
        gath(0, A)

        def body(t, carry):
            k = 2 * t
            drain_g(A)
            wrt(k, A)
            drain_w(B)
            gath(k + 1, B)
            drain_g(B)
            wrt(k + 1, B)
            drain_w(A)
            gath(k + 2, A)
            return carry

        lax.fori_loop(0, (nchunk - 1) // 2, body, 0)
        if nchunk % 2:
            drain_g(A)
            wrt(nchunk - 1, A)
            drain_w(B)
            drain_w(A)
        else:
            drain_g(A)
            wrt(nchunk - 2, A)
            drain_w(B)
            gath(nchunk - 1, B)
            drain_g(B)
            wrt(nchunk - 1, B)
            drain_w(A)
            drain_w(B)

    return body_fn


def _make_scatter_body(epw, nchunk):
    def body_fn(dst_hbm, ea_hbm, zeros_hbm, out_hbm, idx_v, rows_a,
                rows_b, acc, seml_a, seml_b):
        c = lax.axis_index("c")
        s = lax.axis_index("s")
        row0 = s * RPT
        pltpu.sync_copy(zeros_hbm, acc.at[pl.ds(row0, RPT)])
        wid = c * NS + s
        base = wid * epw
        pltpu.sync_copy(dst_hbm.at[wid], idx_v)
        plsc.subcore_barrier()

        def load(k, buf, seml):
            pltpu.async_copy(ea_hbm.at[pl.ds(base + k * CHUNK, CHUNK)], buf,
                             seml)

        def drain(buf, seml):
            pltpu.make_async_copy(ea_hbm.at[pl.ds(base, CHUNK)], buf,
                                  seml).wait()

        def scat(k, buf):
            pltpu.sync_copy(buf, acc.at[idx_v.at[k]], add=True)

        npairs = (nchunk - 1) // 2
        load(0, rows_a, seml_a)

        def body(t, carry):
            k = 2 * t
            drain(rows_a, seml_a)
            load(k + 1, rows_b, seml_b)
            scat(k, rows_a)
            drain(rows_b, seml_b)
            load(k + 2, rows_a, seml_a)
            scat(k + 1, rows_b)
            return carry

        lax.fori_loop(0, npairs, body, 0)
        if nchunk % 2:
            drain(rows_a, seml_a)
            scat(nchunk - 1, rows_a)
        else:
            drain(rows_a, seml_a)
            load(nchunk - 1, rows_b, seml_b)
            scat(nchunk - 2, rows_a)
            drain(rows_b, seml_b)
            scat(nchunk - 1, rows_b)
        plsc.subcore_barrier()
        pltpu.sync_copy(acc.at[pl.ds(row0, RPT)],
                        out_hbm.at[c, pl.ds(row0, RPT)])

    return body_fn


@functools.cache
def _sc_kernels(eh):
    epw = eh // NW
    nchunk = epw // CHUNK
    mesh = plsc.VectorSubcoreMesh(core_axis_name="c", subcore_axis_name="s",
                                  num_cores=NC, num_subcores=NS)
    gather = pl.kernel(
        _make_gather_body(epw, nchunk),
        out_type=(jax.ShapeDtypeStruct((eh, D), jnp.float32),
                  jax.ShapeDtypeStruct((eh, D), jnp.float32)),
        mesh=mesh,
        scratch_types=[
            pltpu.VMEM((nchunk, CHUNK), jnp.int32),
            pltpu.VMEM((nchunk, CHUNK), jnp.int32),
            pltpu.VMEM((CHUNK, D), jnp.float32),
            pltpu.VMEM((CHUNK, D), jnp.float32),
            pltpu.VMEM((CHUNK, D), jnp.float32),
            pltpu.VMEM((CHUNK, D), jnp.float32),
            pltpu.SemaphoreType.DMA,
            pltpu.SemaphoreType.DMA,
            pltpu.SemaphoreType.DMA,
            pltpu.SemaphoreType.DMA,
        ],
    )
    scatter = pl.kernel(
        _make_scatter_body(epw, nchunk),
        out_type=jax.ShapeDtypeStruct((NC, NPAD, D), jnp.float32),
        mesh=mesh,
        scratch_types=[
            pltpu.VMEM((nchunk, CHUNK), jnp.int32),
            pltpu.VMEM((CHUNK, D), jnp.float32),
            pltpu.VMEM((CHUNK, D), jnp.float32),
            pltpu.VMEM_SHARED((NPAD, D), jnp.float32),
            pltpu.SemaphoreType.DMA,
            pltpu.SemaphoreType.DMA,
        ],
    )
    return gather, scatter


def _gather_sc(h, src, dst, xs, xd):
    return _sc_kernels(EH[h])[0](src, dst, xs, xd)


def _scatter_sc(h, dst, ea, zeros):
    return _sc_kernels(EH[h])[1](dst, ea, zeros)


def _full(shape):
    return pl.BlockSpec(shape, lambda i: (0,) * len(shape))


def _xsxd_body(x_ref, w1s_ref, w1d_ref, b1_ref, xs_ref, xd_ref):
    x = x_ref[...]
    xs_ref[...] = jnp.dot(x, w1s_ref[...], preferred_element_type=jnp.float32)
    xd_ref[...] = (jnp.dot(x, w1d_ref[...], preferred_element_type=jnp.float32)
                   + b1_ref[...])


def _ln(h, g, b):
    mu = jnp.mean(h, axis=-1, keepdims=True)
    hc = h - mu
    var = jnp.mean(hc * hc, axis=-1, keepdims=True)
    return hc * lax.rsqrt(var + 1e-5) * g + b


def _edge_body(gs_ref, gd_ref, ea_ref, w1e_ref, w2_ref, b2_ref, w3_ref,
               b3_ref, g_ref, b_ref, out_ref):
    ea = ea_ref[...]
    bf = jnp.bfloat16
    h = (gs_ref[...] + gd_ref[...]
         + jnp.dot(ea.astype(bf), w1e_ref[...].astype(bf),
                   preferred_element_type=jnp.float32))
    h = jnp.maximum(h, 0.0)
    h = jnp.dot(h.astype(bf), w2_ref[...].astype(bf),
                preferred_element_type=jnp.float32) + b2_ref[...]
    h = jnp.maximum(h, 0.0)
    h = jnp.dot(h.astype(bf), w3_ref[...].astype(bf),
                preferred_element_type=jnp.float32) + b3_ref[...]
    out_ref[...] = _ln(h, g_ref[...], b_ref[...]) + ea


def _node_body(x_ref, p0_ref, p1_ref, w1x_ref, w1a_ref, b1_ref, w2_ref,
               b2_ref, w3_ref, b3_ref, g_ref, b_ref, out_ref):
    x = x_ref[...]
    agg = (p0_ref[0] + p0_ref[1]) + (p1_ref[0] + p1_ref[1])
    h = (jnp.dot(x, w1x_ref[...], preferred_element_type=jnp.float32)
         + jnp.dot(agg, w1a_ref[...], preferred_element_type=jnp.float32)
         + b1_ref[...])
    h = jnp.maximum(h, 0.0)
    h = jnp.dot(h, w2_ref[...], preferred_element_type=jnp.float32) + b2_ref[...]
    h = jnp.maximum(h, 0.0)
    h = jnp.dot(h, w3_ref[...], preferred_element_type=jnp.float32) + b3_ref[...]
    out_ref[...] = _ln(h, g_ref[...], b_ref[...]) + x


TN = 1000
TE = 2560


def _xsxd_call(x, w1s, w1d, b1):
    return pl.pallas_call(
        _xsxd_body,
        grid=(N // TN,),
        in_specs=[pl.BlockSpec((TN, D), lambda i: (i, 0)),
                  _full((D, D)), _full((D, D)), _full((1, D))],
        out_specs=[pl.BlockSpec((TN, D), lambda i: (i, 0))] * 2,
        out_shape=[jax.ShapeDtypeStruct((N, D), jnp.float32)] * 2,
    )(x, w1s, w1d, b1)


def _edge_call(eh, ea_off, gs, gd, ea, w1e, w2, b2, w3, b3, g, b):
    row = pl.BlockSpec((TE, D), lambda i: (i, 0))
    ob = ea_off // TE
    row_ea = pl.BlockSpec((TE, D), lambda i: (ob + i, 0))
    return pl.pallas_call(
        _edge_body,
        grid=(eh // TE,),
        in_specs=[row, row, row_ea, _full((D, D)), _full((D, D)),
                  _full((1, D)), _full((D, D)), _full((1, D)), _full((1, D)),
                  _full((1, D))],
        out_specs=row,
        out_shape=jax.ShapeDtypeStruct((eh, D), jnp.float32),
    )(gs, gd, ea, w1e, w2, b2, w3, b3, g, b)


def _node_call(x, p0, p1, w1x, w1a, b1, w2, b2, w3, b3, g, b):
    row = pl.BlockSpec((TN, D), lambda i: (i, 0))
    agg_spec = pl.BlockSpec((NC, TN, D), lambda i: (0, i, 0))
    return pl.pallas_call(
        _node_body,
        grid=(N // TN,),
        in_specs=[row, agg_spec, agg_spec,
                  _full((D, D)), _full((D, D)), _full((1, D)),
                  _full((D, D)), _full((1, D)),
                  _full((D, D)), _full((1, D)), _full((1, D)), _full((1, D))],
        out_specs=row,
        out_shape=jax.ShapeDtypeStruct((N, D), jnp.float32),
    )(x, p0, p1, w1x, w1a, b1, w2, b2, w3, b3, g, b)


def kernel(x, edge_index, edge_attr, params):
    src_h, dst_h = [], []
    for h in range(2):
        epw = EH[h] // NW
        nch = epw // CHUNK
        sl = edge_index[:, EOFF[h]:EOFF[h] + EH[h]]
        src_h.append(sl[0].reshape(NW, nch, CHUNK))
        dst_h.append(sl[1].reshape(NW, nch, CHUNK))
    zeros = jnp.zeros((RPT, D), jnp.float32)
    r = lambda v: v.reshape(1, D)
    ea = (edge_attr, edge_attr)
    ea_off = (EOFF[0], EOFF[1])
    for blk in params:
        (W1, b1), (W2, b2), (W3, b3) = blk["edge"]["linears"]
        xs, xd = _xsxd_call(x, W1[:D], W1[D:2 * D], r(b1))
        g0 = _gather_sc(0, src_h[0], dst_h[0], xs, xd)
        g1 = _gather_sc(1, src_h[1], dst_h[1], xs, xd)
        eargs = (W1[2 * D:], W2, r(b2), W3, r(b3), r(blk["edge"]["ln_g"]),
                 r(blk["edge"]["ln_b"]))
        ea0 = _edge_call(EH[0], ea_off[0], g0[0], g0[1], ea[0], *eargs)
        p0 = _scatter_sc(0, dst_h[0], ea0, zeros)
        ea1 = _edge_call(EH[1], ea_off[1], g1[0], g1[1], ea[1], *eargs)
        p1 = _scatter_sc(1, dst_h[1], ea1, zeros)
        ea, ea_off = (ea0, ea1), (0, 0)
        (V1, c1), (V2, c2), (V3, c3) = blk["node"]["linears"]
        x = _node_call(x, p0, p1, V1[:D], V1[D:], r(c1), V2, r(c2), V3, r(c3),
                       r(blk["node"]["ln_g"]), r(blk["node"]["ln_b"]))
    return x


# speedup vs baseline: 4.2526x; 1.0043x over previous
import functools

import jax
import jax.numpy as jnp
from jax import lax
from jax.experimental import pallas as pl
from jax.experimental.pallas import tpu as pltpu
from jax.experimental.pallas import tpu_sc as plsc

N = 10000
E = 320000
D = 128

NC = 2
NS = 16
NW = NC * NS
CHUNK = 80
NPAD = 10240
RPT = NPAD // NS

EH = (192000, 128000)
EOFF = (0, 192000)


def _make_gather_body(epw, nchunk):
    def body_fn(src_hbm, dst_hbm, xs_hbm, xd_hbm, outs_hbm, outd_hbm,
                idx_s, idx_d, rs_a, rd_a, rs_b, rd_b,
                semg_a, semg_b, semw_a, semw_b):
        wid = lax.axis_index("s") * NC + lax.axis_index("c")
        base = wid * epw
        pltpu.sync_copy(src_hbm.at[wid], idx_s)
        pltpu.sync_copy(dst_hbm.at[wid], idx_d)

        A = (rs_a, rd_a, semg_a, semw_a)
        B = (rs_b, rd_b, semg_b, semw_b)

        def gath(k, buf):
            rs, rd, semg, _ = buf
            pltpu.async_copy(xs_hbm.at[idx_s.at[k]], rs, semg)
            pltpu.async_copy(xd_hbm.at[idx_d.at[k]], rd, semg)

        def drain_g(buf):
            rs, rd, semg, _ = buf
            pltpu.make_async_copy(xs_hbm.at[pl.ds(0, CHUNK)], rs, semg).wait()
            pltpu.make_async_copy(xs_hbm.at[pl.ds(0, CHUNK)], rd, semg).wait()

        def wrt(k, buf):
            rs, rd, _, semw = buf
            off = base + k * CHUNK
            pltpu.async_copy(rs, outs_hbm.at[pl.ds(off, CHUNK)], semw)
            pltpu.async_copy(rd, outd_hbm.at[pl.ds(off, CHUNK)], semw)

        def drain_w(buf):
            rs, rd, _, semw = buf
            pltpu.make_async_copy(outs_hbm.at[pl.ds(base, CHUNK)], rs,
                                  semw).wait()
            pltpu.make_async_copy(outs_hbm.at[pl.ds(base, CHUNK)], rd,
                                  semw).wait()

        wrt(1, B)
        gath(0, A)

        def body(t, carry):
            k = 2 * t
            drain_g(A)
            wrt(k, A)
            drain_w(B)
            gath(k + 1, B)
            drain_g(B)
            wrt(k + 1, B)
            drain_w(A)
            gath(k + 2, A)
            return carry

        lax.fori_loop(0, (nchunk - 1) // 2, body, 0)
        if nchunk % 2:
            drain_g(A)
            wrt(nchunk - 1, A)
            drain_w(B)
            drain_w(A)
        else:
            drain_g(A)
            wrt(nchunk - 2, A)
            drain_w(B)
            gath(nchunk - 1, B)
            drain_g(B)
            wrt(nchunk - 1, B)
            drain_w(A)
            drain_w(B)

    return body_fn


def _make_scatter_body(epw, nchunk):
    def body_fn(dst_hbm, ea_hbm, zeros_hbm, out_hbm, idx_v, rows_a,
                rows_b, acc, seml_a, seml_b):
        c = lax.axis_index("c")
        s = lax.axis_index("s")
        row0 = s * RPT
        pltpu.sync_copy(zeros_hbm, acc.at[pl.ds(row0, RPT)])
        wid = c * NS + s
        base = wid * epw
        pltpu.sync_copy(dst_hbm.at[wid], idx_v)
        plsc.subcore_barrier()

        def load(k, buf, seml):
            pltpu.async_copy(ea_hbm.at[pl.ds(base + k * CHUNK, CHUNK)], buf,
                             seml)

        def drain(buf, seml):
            pltpu.make_async_copy(ea_hbm.at[pl.ds(base, CHUNK)], buf,
                                  seml).wait()

        def scat(k, buf):
            pltpu.sync_copy(buf, acc.at[idx_v.at[k]], add=True)

        npairs = (nchunk - 1) // 2
        load(0, rows_a, seml_a)

        def body(t, carry):
            k = 2 * t
            drain(rows_a, seml_a)
            load(k + 1, rows_b, seml_b)
            scat(k, rows_a)
            drain(rows_b, seml_b)
            load(k + 2, rows_a, seml_a)
            scat(k + 1, rows_b)
            return carry

        lax.fori_loop(0, npairs, body, 0)
        if nchunk % 2:
            drain(rows_a, seml_a)
            scat(nchunk - 1, rows_a)
        else:
            drain(rows_a, seml_a)
            load(nchunk - 1, rows_b, seml_b)
            scat(nchunk - 2, rows_a)
            drain(rows_b, seml_b)
            scat(nchunk - 1, rows_b)
        plsc.subcore_barrier()
        pltpu.sync_copy(acc.at[pl.ds(row0, RPT)],
                        out_hbm.at[c, pl.ds(row0, RPT)])

    return body_fn


@functools.cache
def _sc_kernels(eh):
    epw = eh // NW
    nchunk = epw // CHUNK
    mesh = plsc.VectorSubcoreMesh(core_axis_name="c", subcore_axis_name="s",
                                  num_cores=NC, num_subcores=NS)
    gather = pl.kernel(
        _make_gather_body(epw, nchunk),
        out_type=(jax.ShapeDtypeStruct((eh, D), jnp.float32),
                  jax.ShapeDtypeStruct((eh, D), jnp.float32)),
        mesh=mesh,
        scratch_types=[
            pltpu.VMEM((nchunk, CHUNK), jnp.int32),
            pltpu.VMEM((nchunk, CHUNK), jnp.int32),
            pltpu.VMEM((CHUNK, D), jnp.float32),
            pltpu.VMEM((CHUNK, D), jnp.float32),
            pltpu.VMEM((CHUNK, D), jnp.float32),
            pltpu.VMEM((CHUNK, D), jnp.float32),
            pltpu.SemaphoreType.DMA,
            pltpu.SemaphoreType.DMA,
            pltpu.SemaphoreType.DMA,
            pltpu.SemaphoreType.DMA,
        ],
    )
    scatter = pl.kernel(
        _make_scatter_body(epw, nchunk),
        out_type=jax.ShapeDtypeStruct((NC, NPAD, D), jnp.float32),
        mesh=mesh,
        scratch_types=[
            pltpu.VMEM((nchunk, CHUNK), jnp.int32),
            pltpu.VMEM((CHUNK, D), jnp.float32),
            pltpu.VMEM((CHUNK, D), jnp.float32),
            pltpu.VMEM_SHARED((NPAD, D), jnp.float32),
            pltpu.SemaphoreType.DMA,
            pltpu.SemaphoreType.DMA,
        ],
    )
    return gather, scatter


def _gather_sc(h, src, dst, xs, xd):
    return _sc_kernels(EH[h])[0](src, dst, xs, xd)


def _scatter_sc(h, dst, ea, zeros):
    return _sc_kernels(EH[h])[1](dst, ea, zeros)


def _full(shape):
    return pl.BlockSpec(shape, lambda i: (0,) * len(shape))


def _xsxd_body(x_ref, w1s_ref, w1d_ref, b1_ref, xs_ref, xd_ref):
    x = x_ref[...]
    xs_ref[...] = jnp.dot(x, w1s_ref[...], preferred_element_type=jnp.float32)
    xd_ref[...] = (jnp.dot(x, w1d_ref[...], preferred_element_type=jnp.float32)
                   + b1_ref[...])


def _ln(h, g, b):
    mu = jnp.mean(h, axis=-1, keepdims=True)
    hc = h - mu
    var = jnp.mean(hc * hc, axis=-1, keepdims=True)
    return hc * lax.rsqrt(var + 1e-5) * g + b


def _edge_body(gs_ref, gd_ref, ea_ref, w1e_ref, w2_ref, b2_ref, w3_ref,
               b3_ref, g_ref, b_ref, out_ref):
    ea = ea_ref[...]
    bf = jnp.bfloat16
    h = (gs_ref[...] + gd_ref[...]
         + jnp.dot(ea.astype(bf), w1e_ref[...].astype(bf),
                   preferred_element_type=jnp.float32))
    h = jnp.maximum(h, 0.0)
    h = jnp.dot(h.astype(bf), w2_ref[...].astype(bf),
                preferred_element_type=jnp.float32) + b2_ref[...]
    h = jnp.maximum(h, 0.0)
    h = jnp.dot(h.astype(bf), w3_ref[...].astype(bf),
                preferred_element_type=jnp.float32) + b3_ref[...]
    out_ref[...] = _ln(h, g_ref[...], b_ref[...]) + ea


def _node_body(x_ref, p0_ref, p1_ref, w1x_ref, w1a_ref, b1_ref, w2_ref,
               b2_ref, w3_ref, b3_ref, g_ref, b_ref, out_ref):
    x = x_ref[...]
    agg = (p0_ref[0] + p0_ref[1]) + (p1_ref[0] + p1_ref[1])
    h = (jnp.dot(x, w1x_ref[...], preferred_element_type=jnp.float32)
         + jnp.dot(agg, w1a_ref[...], preferred_element_type=jnp.float32)
         + b1_ref[...])
    h = jnp.maximum(h, 0.0)
    h = jnp.dot(h, w2_ref[...], preferred_element_type=jnp.float32) + b2_ref[...]
    h = jnp.maximum(h, 0.0)
    h = jnp.dot(h, w3_ref[...], preferred_element_type=jnp.float32) + b3_ref[...]
    out_ref[...] = _ln(h, g_ref[...], b_ref[...]) + x


TN = 1000
TE = 3200


def _xsxd_call(x, w1s, w1d, b1):
    return pl.pallas_call(
        _xsxd_body,
        grid=(N // TN,),
        in_specs=[pl.BlockSpec((TN, D), lambda i: (i, 0)),
                  _full((D, D)), _full((D, D)), _full((1, D))],
        out_specs=[pl.BlockSpec((TN, D), lambda i: (i, 0))] * 2,
        out_shape=[jax.ShapeDtypeStruct((N, D), jnp.float32)] * 2,
    )(x, w1s, w1d, b1)


def _edge_call(eh, ea_off, gs, gd, ea, w1e, w2, b2, w3, b3, g, b):
    row = pl.BlockSpec((TE, D), lambda i: (i, 0))
    ob = ea_off // TE
    row_ea = pl.BlockSpec((TE, D), lambda i: (ob + i, 0))
    return pl.pallas_call(
        _edge_body,
        grid=(eh // TE,),
        in_specs=[row, row, row_ea, _full((D, D)), _full((D, D)),
                  _full((1, D)), _full((D, D)), _full((1, D)), _full((1, D)),
                  _full((1, D))],
        out_specs=row,
        out_shape=jax.ShapeDtypeStruct((eh, D), jnp.float32),
    )(gs, gd, ea, w1e, w2, b2, w3, b3, g, b)


def _node_call(x, p0, p1, w1x, w1a, b1, w2, b2, w3, b3, g, b):
    row = pl.BlockSpec((TN, D), lambda i: (i, 0))
    agg_spec = pl.BlockSpec((NC, TN, D), lambda i: (0, i, 0))
    return pl.pallas_call(
        _node_body,
        grid=(N // TN,),
        in_specs=[row, agg_spec, agg_spec,
                  _full((D, D)), _full((D, D)), _full((1, D)),
                  _full((D, D)), _full((1, D)),
                  _full((D, D)), _full((1, D)), _full((1, D)), _full((1, D))],
        out_specs=row,
        out_shape=jax.ShapeDtypeStruct((N, D), jnp.float32),
    )(x, p0, p1, w1x, w1a, b1, w2, b2, w3, b3, g, b)


def kernel(x, edge_index, edge_attr, params):
    src_h, dst_h = [], []
    for h in range(2):
        epw = EH[h] // NW
        nch = epw // CHUNK
        sl = edge_index[:, EOFF[h]:EOFF[h] + EH[h]]
        src_h.append(sl[0].reshape(NW, nch, CHUNK))
        dst_h.append(sl[1].reshape(NW, nch, CHUNK))
    zeros = jnp.zeros((RPT, D), jnp.float32)
    r = lambda v: v.reshape(1, D)
    ea = (edge_attr, edge_attr)
    ea_off = (EOFF[0], EOFF[1])
    for blk in params:
        (W1, b1), (W2, b2), (W3, b3) = blk["edge"]["linears"]
        xs, xd = _xsxd_call(x, W1[:D], W1[D:2 * D], r(b1))
        g0 = _gather_sc(0, src_h[0], dst_h[0], xs, xd)
        g1 = _gather_sc(1, src_h[1], dst_h[1], xs, xd)
        eargs = (W1[2 * D:], W2, r(b2), W3, r(b3), r(blk["edge"]["ln_g"]),
                 r(blk["edge"]["ln_b"]))
        ea0 = _edge_call(EH[0], ea_off[0], g0[0], g0[1], ea[0], *eargs)
        p0 = _scatter_sc(0, dst_h[0], ea0, zeros)
        ea1 = _edge_call(EH[1], ea_off[1], g1[0], g1[1], ea[1], *eargs)
        p1 = _scatter_sc(1, dst_h[1], ea1, zeros)
        ea, ea_off = (ea0, ea1), (0, 0)
        (V1, c1), (V2, c2), (V3, c3) = blk["node"]["linears"]
        x = _node_call(x, p0, p1, V1[:D], V1[D:], r(c1), V2, r(c2), V3, r(c3),
                       r(blk["node"]["ln_g"]), r(blk["node"]["ln_b"]))
    return x


# speedup vs baseline: 4.2695x; 1.0040x over previous
import functools

import jax
import jax.numpy as jnp
from jax import lax
from jax.experimental import pallas as pl
from jax.experimental.pallas import tpu as pltpu
from jax.experimental.pallas import tpu_sc as plsc

N = 10000
E = 320000
D = 128

NC = 2
NS = 16
NW = NC * NS
CHUNK = 80
NPAD = 10240
RPT = NPAD // NS

EH = (192000, 128000)
EOFF = (0, 192000)


def _make_gather_body(epw, nchunk):
    def body_fn(src_hbm, dst_hbm, xs_hbm, xd_hbm, outs_hbm, outd_hbm,
                idx_s, idx_d, rs_a, rd_a, rs_b, rd_b,
                semg_a, semg_b, semw_a, semw_b):
        wid = lax.axis_index("s") * NC + lax.axis_index("c")
        base = wid * epw
        pltpu.sync_copy(src_hbm.at[wid], idx_s)
        pltpu.sync_copy(dst_hbm.at[wid], idx_d)

        A = (rs_a, rd_a, semg_a, semw_a)
        B = (rs_b, rd_b, semg_b, semw_b)

        def gath(k, buf):
            rs, rd, semg, _ = buf
            pltpu.async_copy(xs_hbm.at[idx_s.at[k]], rs, semg)
            pltpu.async_copy(xd_hbm.at[idx_d.at[k]], rd, semg)

        def drain_g(buf):
            rs, rd, semg, _ = buf
            pltpu.make_async_copy(xs_hbm.at[pl.ds(0, CHUNK)], rs, semg).wait()
            pltpu.make_async_copy(xs_hbm.at[pl.ds(0, CHUNK)], rd, semg).wait()

        def wrt(k, buf):
            rs, rd, _, semw = buf
            off = base + k * CHUNK
            pltpu.async_copy(rs, outs_hbm.at[pl.ds(off, CHUNK)], semw)
            pltpu.async_copy(rd, outd_hbm.at[pl.ds(off, CHUNK)], semw)

        def drain_w(buf):
            rs, rd, _, semw = buf
            pltpu.make_async_copy(outs_hbm.at[pl.ds(base, CHUNK)], rs,
                                  semw).wait()
            pltpu.make_async_copy(outs_hbm.at[pl.ds(base, CHUNK)], rd,
                                  semw).wait()

        wrt(1, B)
        gath(0, A)

        def body(t, carry):
            k = 2 * t
            drain_g(A)
            wrt(k, A)
            drain_w(B)
            gath(k + 1, B)
            drain_g(B)
            wrt(k + 1, B)
            drain_w(A)
            gath(k + 2, A)
            return carry

        lax.fori_loop(0, (nchunk - 1) // 2, body, 0)
        if nchunk % 2:
            drain_g(A)
            wrt(nchunk - 1, A)
            drain_w(B)
            drain_w(A)
        else:
            drain_g(A)
            wrt(nchunk - 2, A)
            drain_w(B)
            gath(nchunk - 1, B)
            drain_g(B)
            wrt(nchunk - 1, B)
            drain_w(A)
            drain_w(B)

    return body_fn


def _make_scatter_body(epw, nchunk):
    def body_fn(dst_hbm, ea_hbm, zeros_hbm, out_hbm, idx_v, rows_a,
                rows_b, acc, seml_a, seml_b):
        c = lax.axis_index("c")
        s = lax.axis_index("s")
        row0 = s * RPT
        pltpu.sync_copy(zeros_hbm, acc.at[pl.ds(row0, RPT)])
        wid = c * NS + s
        base = wid * epw
        pltpu.sync_copy(dst_hbm.at[wid], idx_v)
        plsc.subcore_barrier()

        def load(k, buf, seml):
            pltpu.async_copy(ea_hbm.at[pl.ds(base + k * CHUNK, CHUNK)], buf,
                             seml)

        def drain(buf, seml):
            pltpu.make_async_copy(ea_hbm.at[pl.ds(base, CHUNK)], buf,
                                  seml).wait()

        def scat(k, buf):
            pltpu.sync_copy(buf, acc.at[idx_v.at[k]], add=True)

        npairs = (nchunk - 1) // 2
        load(0, rows_a, seml_a)

        def body(t, carry):
            k = 2 * t
            drain(rows_a, seml_a)
            load(k + 1, rows_b, seml_b)
            scat(k, rows_a)
            drain(rows_b, seml_b)
            load(k + 2, rows_a, seml_a)
            scat(k + 1, rows_b)
            return carry

        lax.fori_loop(0, npairs, body, 0)
        if nchunk % 2:
            drain(rows_a, seml_a)
            scat(nchunk - 1, rows_a)
        else:
            drain(rows_a, seml_a)
            load(nchunk - 1, rows_b, seml_b)
            scat(nchunk - 2, rows_a)
            drain(rows_b, seml_b)
            scat(nchunk - 1, rows_b)
        plsc.subcore_barrier()
        pltpu.sync_copy(acc.at[pl.ds(row0, RPT)],
                        out_hbm.at[c, pl.ds(row0, RPT)])

    return body_fn


@functools.cache
def _sc_kernels(eh):
    epw = eh // NW
    nchunk = epw // CHUNK
    mesh = plsc.VectorSubcoreMesh(core_axis_name="c", subcore_axis_name="s",
                                  num_cores=NC, num_subcores=NS)
    gather = pl.kernel(
        _make_gather_body(epw, nchunk),
        out_type=(jax.ShapeDtypeStruct((eh, D), jnp.float32),
                  jax.ShapeDtypeStruct((eh, D), jnp.float32)),
        mesh=mesh,
        scratch_types=[
            pltpu.VMEM((nchunk, CHUNK), jnp.int32),
            pltpu.VMEM((nchunk, CHUNK), jnp.int32),
            pltpu.VMEM((CHUNK, D), jnp.float32),
            pltpu.VMEM((CHUNK, D), jnp.float32),
            pltpu.VMEM((CHUNK, D), jnp.float32),
            pltpu.VMEM((CHUNK, D), jnp.float32),
            pltpu.SemaphoreType.DMA,
            pltpu.SemaphoreType.DMA,
            pltpu.SemaphoreType.DMA,
            pltpu.SemaphoreType.DMA,
        ],
    )
    scatter = pl.kernel(
        _make_scatter_body(epw, nchunk),
        out_type=jax.ShapeDtypeStruct((NC, NPAD, D), jnp.float32),
        mesh=mesh,
        scratch_types=[
            pltpu.VMEM((nchunk, CHUNK), jnp.int32),
            pltpu.VMEM((CHUNK, D), jnp.float32),
            pltpu.VMEM((CHUNK, D), jnp.float32),
            pltpu.VMEM_SHARED((NPAD, D), jnp.float32),
            pltpu.SemaphoreType.DMA,
            pltpu.SemaphoreType.DMA,
        ],
    )
    return gather, scatter


def _gather_sc(h, src, dst, xs, xd):
    return _sc_kernels(EH[h])[0](src, dst, xs, xd)


def _scatter_sc(h, dst, ea, zeros):
    return _sc_kernels(EH[h])[1](dst, ea, zeros)


def _full(shape):
    return pl.BlockSpec(shape, lambda i: (0,) * len(shape))


def _xsxd_body(x_ref, w1s_ref, w1d_ref, b1_ref, xs_ref, xd_ref):
    x = x_ref[...]
    xs_ref[...] = jnp.dot(x, w1s_ref[...], preferred_element_type=jnp.float32)
    xd_ref[...] = (jnp.dot(x, w1d_ref[...], preferred_element_type=jnp.float32)
                   + b1_ref[...])


def _ln(h, g, b):
    mu = jnp.mean(h, axis=-1, keepdims=True)
    hc = h - mu
    var = jnp.mean(hc * hc, axis=-1, keepdims=True)
    return hc * lax.rsqrt(var + 1e-5) * g + b


def _edge_body(gs_ref, gd_ref, ea_ref, w1e_ref, w2_ref, b2_ref, w3_ref,
               b3_ref, g_ref, b_ref, out_ref):
    ea = ea_ref[...]
    bf = jnp.bfloat16
    h = (gs_ref[...] + gd_ref[...]
         + jnp.dot(ea.astype(bf), w1e_ref[...].astype(bf),
                   preferred_element_type=jnp.float32))
    h = jnp.maximum(h, 0.0)
    h = jnp.dot(h.astype(bf), w2_ref[...].astype(bf),
                preferred_element_type=jnp.float32) + b2_ref[...]
    h = jnp.maximum(h, 0.0)
    h = jnp.dot(h.astype(bf), w3_ref[...].astype(bf),
                preferred_element_type=jnp.float32) + b3_ref[...]
    out_ref[...] = _ln(h, g_ref[...], b_ref[...]) + ea


def _node_body(x_ref, p0_ref, p1_ref, w1x_ref, w1a_ref, b1_ref, w2_ref,
               b2_ref, w3_ref, b3_ref, g_ref, b_ref, out_ref):
    x = x_ref[...]
    agg = (p0_ref[0] + p0_ref[1]) + (p1_ref[0] + p1_ref[1])
    h = (jnp.dot(x, w1x_ref[...], preferred_element_type=jnp.float32)
         + jnp.dot(agg, w1a_ref[...], preferred_element_type=jnp.float32)
         + b1_ref[...])
    h = jnp.maximum(h, 0.0)
    h = jnp.dot(h, w2_ref[...], preferred_element_type=jnp.float32) + b2_ref[...]
    h = jnp.maximum(h, 0.0)
    h = jnp.dot(h, w3_ref[...], preferred_element_type=jnp.float32) + b3_ref[...]
    out_ref[...] = _ln(h, g_ref[...], b_ref[...]) + x


TN = 1000
TE = 6400


def _xsxd_call(x, w1s, w1d, b1):
    return pl.pallas_call(
        _xsxd_body,
        grid=(N // TN,),
        in_specs=[pl.BlockSpec((TN, D), lambda i: (i, 0)),
                  _full((D, D)), _full((D, D)), _full((1, D))],
        out_specs=[pl.BlockSpec((TN, D), lambda i: (i, 0))] * 2,
        out_shape=[jax.ShapeDtypeStruct((N, D), jnp.float32)] * 2,
    )(x, w1s, w1d, b1)


def _edge_call(eh, ea_off, gs, gd, ea, w1e, w2, b2, w3, b3, g, b):
    row = pl.BlockSpec((TE, D), lambda i: (i, 0))
    ob = ea_off // TE
    row_ea = pl.BlockSpec((TE, D), lambda i: (ob + i, 0))
    return pl.pallas_call(
        _edge_body,
        grid=(eh // TE,),
        in_specs=[row, row, row_ea, _full((D, D)), _full((D, D)),
                  _full((1, D)), _full((D, D)), _full((1, D)), _full((1, D)),
                  _full((1, D))],
        out_specs=row,
        out_shape=jax.ShapeDtypeStruct((eh, D), jnp.float32),
    )(gs, gd, ea, w1e, w2, b2, w3, b3, g, b)


def _node_call(x, p0, p1, w1x, w1a, b1, w2, b2, w3, b3, g, b):
    row = pl.BlockSpec((TN, D), lambda i: (i, 0))
    agg_spec = pl.BlockSpec((NC, TN, D), lambda i: (0, i, 0))
    return pl.pallas_call(
        _node_body,
        grid=(N // TN,),
        in_specs=[row, agg_spec, agg_spec,
                  _full((D, D)), _full((D, D)), _full((1, D)),
                  _full((D, D)), _full((1, D)),
                  _full((D, D)), _full((1, D)), _full((1, D)), _full((1, D))],
        out_specs=row,
        out_shape=jax.ShapeDtypeStruct((N, D), jnp.float32),
    )(x, p0, p1, w1x, w1a, b1, w2, b2, w3, b3, g, b)


def kernel(x, edge_index, edge_attr, params):
    src_h, dst_h = [], []
    for h in range(2):
        epw = EH[h] // NW
        nch = epw // CHUNK
        sl = edge_index[:, EOFF[h]:EOFF[h] + EH[h]]
        src_h.append(sl[0].reshape(NW, nch, CHUNK))
        dst_h.append(sl[1].reshape(NW, nch, CHUNK))
    zeros = jnp.zeros((RPT, D), jnp.float32)
    r = lambda v: v.reshape(1, D)
    ea = (edge_attr, edge_attr)
    ea_off = (EOFF[0], EOFF[1])
    for blk in params:
        (W1, b1), (W2, b2), (W3, b3) = blk["edge"]["linears"]
        xs, xd = _xsxd_call(x, W1[:D], W1[D:2 * D], r(b1))
        g0 = _gather_sc(0, src_h[0], dst_h[0], xs, xd)
        g1 = _gather_sc(1, src_h[1], dst_h[1], xs, xd)
        eargs = (W1[2 * D:], W2, r(b2), W3, r(b3), r(blk["edge"]["ln_g"]),
                 r(blk["edge"]["ln_b"]))
        ea0 = _edge_call(EH[0], ea_off[0], g0[0], g0[1], ea[0], *eargs)
        p0 = _scatter_sc(0, dst_h[0], ea0, zeros)
        ea1 = _edge_call(EH[1], ea_off[1], g1[0], g1[1], ea[1], *eargs)
        p1 = _scatter_sc(1, dst_h[1], ea1, zeros)
        ea, ea_off = (ea0, ea1), (0, 0)
        (V1, c1), (V2, c2), (V3, c3) = blk["node"]["linears"]
        x = _node_call(x, p0, p1, V1[:D], V1[D:], r(c1), V2, r(c2), V3, r(c3),
                       r(blk["node"]["ln_g"]), r(blk["node"]["ln_b"]))
    return x
